# baseline (speedup 1.0000x reference)
import functools

import jax
import jax.numpy as jnp
from jax import lax
from jax.experimental import pallas as pl
from jax.experimental.pallas import tpu as pltpu
from jax.experimental.pallas import tpu_sc as plsc

D_MODEL = 1024
SEQ = 4096
BATCH = 4
NUM_WORKERS = 32
S_PER_W = SEQ // NUM_WORKERS
CHUNK_S = 16
CHUNKS = S_PER_W // CHUNK_S
CHUNK_W = CHUNK_S * D_MODEL
NITER = CHUNKS * BATCH

_mesh = plsc.VectorSubcoreMesh(core_axis_name="c", subcore_axis_name="s")


@functools.partial(
    pl.kernel,
    out_type=jax.ShapeDtypeStruct((BATCH * SEQ * D_MODEL,), jnp.float32),
    mesh=_mesh,
    scratch_types=[
        pltpu.VMEM((CHUNK_W,), jnp.float32),
        pltpu.VMEM((CHUNK_W,), jnp.float32),
        pltpu.VMEM((CHUNK_W,), jnp.float32),
        pltpu.VMEM((CHUNK_W,), jnp.float32),
        pltpu.SemaphoreType.DMA,
        pltpu.SemaphoreType.DMA,
        pltpu.SemaphoreType.DMA,
        pltpu.SemaphoreType.DMA,
        pltpu.SemaphoreType.DMA,
        pltpu.SemaphoreType.DMA,
    ],
)
def _sc_add_pe(x_hbm, pe_hbm, out_hbm, pe0, pe1, x0, x1, ps0, ps1, ls0, ls1, ss0, ss1):
    wid = lax.axis_index("s") * 2 + lax.axis_index("c")
    s0 = wid * S_PER_W
    pe_bufs, x_bufs = (pe0, pe1), (x0, x1)
    psems, lsems, ssems = (ps0, ps1), (ls0, ls1), (ss0, ss1)

    def x_off(j):
        i, b = divmod(j, BATCH)
        return (b * SEQ + s0 + i * CHUNK_S) * D_MODEL

    def start_pe(i):
        src = pe_hbm.at[pl.ds((s0 + i * CHUNK_S) * D_MODEL, CHUNK_W)]
        return pltpu.async_copy(src, pe_bufs[i & 1], psems[i & 1])

    def start_load(j):
        src = x_hbm.at[pl.ds(x_off(j), CHUNK_W)]
        return pltpu.async_copy(src, x_bufs[j & 1], lsems[j & 1])

    def start_store(j):
        dst = out_hbm.at[pl.ds(x_off(j), CHUNK_W)]
        return pltpu.async_copy(x_bufs[j & 1], dst, ssems[j & 1])

    pe_d = {0: start_pe(0)}
    ld = {0: start_load(0)}
    st = {}
    for j in range(NITER):
        p = j & 1
        i = j // BATCH
        if j % BATCH == 0:
            pe_d[i].wait()
            if i + 1 < CHUNKS:
                pe_d[i + 1] = start_pe(i + 1)
        ld[j].wait()
        if j + 1 < NITER:
            if j >= 1:
                st[j - 1].wait()
            ld[j + 1] = start_load(j + 1)

        @plsc.parallel_loop(0, CHUNK_W // 16, unroll=16)
        def _(jj, _pe=pe_bufs[i & 1], _x=x_bufs[p]):
            _x[pl.ds(jj * 16, 16)] = _x[pl.ds(jj * 16, 16)] + _pe[pl.ds(jj * 16, 16)]

        st[j] = start_store(j)
    st[NITER - 2].wait()
    st[NITER - 1].wait()


def kernel(x, pe_table):
    out = _sc_add_pe(x.reshape(-1), pe_table.reshape(-1))
    return out.reshape(x.shape)


# speedup vs baseline: 2.6825x; 2.6825x over previous
import functools

import jax
import jax.numpy as jnp
from jax import lax
from jax.experimental import pallas as pl
from jax.experimental.pallas import tpu as pltpu
from jax.experimental.pallas import tpu_sc as plsc

D_MODEL = 1024
SEQ = 4096
BATCH = 4
NUM_WORKERS = 32
S_PER_W = SEQ // NUM_WORKERS
CHUNK_S = 16
CHUNKS = S_PER_W // CHUNK_S
CHUNK_W = CHUNK_S * D_MODEL
NITER = CHUNKS * BATCH

_mesh = plsc.VectorSubcoreMesh(core_axis_name="c", subcore_axis_name="s")


@functools.partial(
    pl.kernel,
    out_type=jax.ShapeDtypeStruct((BATCH * SEQ, D_MODEL), jnp.float32),
    mesh=_mesh,
    compiler_params=pltpu.CompilerParams(use_tc_tiling_on_sc=True),
    scratch_types=[
        pltpu.VMEM((CHUNK_S, D_MODEL), jnp.float32),
        pltpu.VMEM((CHUNK_S, D_MODEL), jnp.float32),
        pltpu.VMEM((CHUNK_S, D_MODEL), jnp.float32),
        pltpu.VMEM((CHUNK_S, D_MODEL), jnp.float32),
        pltpu.SemaphoreType.DMA,
        pltpu.SemaphoreType.DMA,
        pltpu.SemaphoreType.DMA,
        pltpu.SemaphoreType.DMA,
        pltpu.SemaphoreType.DMA,
        pltpu.SemaphoreType.DMA,
    ],
)
def _sc_add_pe(x_hbm, pe_hbm, out_hbm, pe0, pe1, x0, x1, ps0, ps1, ls0, ls1, ss0, ss1):
    wid = lax.axis_index("s") * 2 + lax.axis_index("c")
    s0 = wid * S_PER_W
    pe_bufs, x_bufs = (pe0, pe1), (x0, x1)
    psems, lsems, ssems = (ps0, ps1), (ls0, ls1), (ss0, ss1)

    def x_row(j):
        i, b = divmod(j, BATCH)
        return b * SEQ + s0 + i * CHUNK_S

    def start_pe(i):
        src = pe_hbm.at[pl.ds(s0 + i * CHUNK_S, CHUNK_S), :]
        return pltpu.async_copy(src, pe_bufs[i & 1], psems[i & 1])

    def start_load(j):
        src = x_hbm.at[pl.ds(x_row(j), CHUNK_S), :]
        return pltpu.async_copy(src, x_bufs[j & 1], lsems[j & 1])

    def start_store(j):
        dst = out_hbm.at[pl.ds(x_row(j), CHUNK_S), :]
        return pltpu.async_copy(x_bufs[j & 1], dst, ssems[j & 1])

    pe_d = {0: start_pe(0)}
    ld = {0: start_load(0)}
    st = {}
    for j in range(NITER):
        p = j & 1
        i = j // BATCH
        if j % BATCH == 0:
            pe_d[i].wait()
            if i + 1 < CHUNKS:
                pe_d[i + 1] = start_pe(i + 1)
        ld[j].wait()
        if j + 1 < NITER:
            if j >= 1:
                st[j - 1].wait()
            ld[j + 1] = start_load(j + 1)

        @plsc.parallel_loop(0, CHUNK_W // 16, unroll=16)
        def _(jj, _pe=pe_bufs[i & 1], _x=x_bufs[p]):
            r, c = divmod(jj * 16, D_MODEL)
            _x[r, pl.ds(c, 16)] = _x[r, pl.ds(c, 16)] + _pe[r, pl.ds(c, 16)]

        st[j] = start_store(j)
    st[NITER - 2].wait()
    st[NITER - 1].wait()


def kernel(x, pe_table):
    out = _sc_add_pe(x.reshape(BATCH * SEQ, D_MODEL), pe_table)
    return out.reshape(x.shape)


# speedup vs baseline: 2.6872x; 1.0017x over previous
import functools

import jax
import jax.numpy as jnp
from jax import lax
from jax.experimental import pallas as pl
from jax.experimental.pallas import tpu as pltpu
from jax.experimental.pallas import tpu_sc as plsc

D_MODEL = 1024
SEQ = 4096
BATCH = 4
NUM_WORKERS = 32
S_PER_W = SEQ // NUM_WORKERS
CHUNK_S = 16
CHUNKS = S_PER_W // CHUNK_S
CHUNK_W = CHUNK_S * D_MODEL
NITER = CHUNKS * BATCH

_mesh = plsc.VectorSubcoreMesh(core_axis_name="c", subcore_axis_name="s")


@functools.partial(
    pl.kernel,
    out_type=jax.ShapeDtypeStruct((BATCH * SEQ, D_MODEL), jnp.float32),
    mesh=_mesh,
    compiler_params=pltpu.CompilerParams(use_tc_tiling_on_sc=True),
    scratch_types=[
        pltpu.VMEM((CHUNK_S, D_MODEL), jnp.float32),
        pltpu.VMEM((CHUNK_S, D_MODEL), jnp.float32),
        pltpu.VMEM((CHUNK_S, D_MODEL), jnp.float32),
        pltpu.VMEM((CHUNK_S, D_MODEL), jnp.float32),
        pltpu.SemaphoreType.DMA,
        pltpu.SemaphoreType.DMA,
        pltpu.SemaphoreType.DMA,
        pltpu.SemaphoreType.DMA,
        pltpu.SemaphoreType.DMA,
        pltpu.SemaphoreType.DMA,
    ],
)
def _sc_add_pe(x_hbm, pe_hbm, out_hbm, pe0, pe1, x0, x1, ps0, ps1, ls0, ls1, ss0, ss1):
    wid = lax.axis_index("s") * 2 + lax.axis_index("c")
    s0 = wid * S_PER_W
    pe_bufs, x_bufs = (pe0, pe1), (x0, x1)
    psems, lsems, ssems = (ps0, ps1), (ls0, ls1), (ss0, ss1)

    def x_row(j):
        i, b = divmod(j, BATCH)
        return b * SEQ + s0 + i * CHUNK_S

    def start_pe(i):
        src = pe_hbm.at[pl.ds(s0 + i * CHUNK_S, CHUNK_S), :]
        return pltpu.async_copy(src, pe_bufs[i & 1], psems[i & 1])

    def start_load(j):
        src = x_hbm.at[pl.ds(x_row(j), CHUNK_S), :]
        return pltpu.async_copy(src, x_bufs[j & 1], lsems[j & 1])

    def start_store(j):
        dst = out_hbm.at[pl.ds(x_row(j), CHUNK_S), :]
        return pltpu.async_copy(x_bufs[j & 1], dst, ssems[j & 1])

    pe_d = {0: start_pe(0)}
    ld = {0: start_load(0)}
    st = {}
    for j in range(NITER):
        p = j & 1
        i = j // BATCH
        if j % BATCH == 0:
            pe_d[i].wait()
            if i + 1 < CHUNKS:
                pe_d[i + 1] = start_pe(i + 1)
        ld[j].wait()
        if j + 1 < NITER:
            if j >= 1:
                st[j - 1].wait()
            ld[j + 1] = start_load(j + 1)

        @plsc.parallel_loop(0, CHUNK_W // 16, unroll=16)
        def _(jj, _pe=pe_bufs[i & 1], _x=x_bufs[p]):
            r, c = divmod(jj * 16, D_MODEL)
            plsc.addupdate(_x.at[r, pl.ds(c, 16)], _pe[r, pl.ds(c, 16)])

        st[j] = start_store(j)
    st[NITER - 2].wait()
    st[NITER - 1].wait()


def kernel(x, pe_table):
    out = _sc_add_pe(x.reshape(BATCH * SEQ, D_MODEL), pe_table)
    return out.reshape(x.shape)


# speedup vs baseline: 2.7661x; 1.0294x over previous
import functools

import jax
import jax.numpy as jnp
from jax import lax
from jax.experimental import pallas as pl
from jax.experimental.pallas import tpu as pltpu
from jax.experimental.pallas import tpu_sc as plsc

D_MODEL = 1024
SEQ = 4096
BATCH = 4
NUM_WORKERS = 32
S_PER_W = SEQ // NUM_WORKERS
CHUNK_S = 16
CHUNKS = S_PER_W // CHUNK_S
CHUNK_W = CHUNK_S * D_MODEL
NITER = CHUNKS * BATCH

_mesh = plsc.VectorSubcoreMesh(core_axis_name="c", subcore_axis_name="s")


@functools.partial(
    pl.kernel,
    out_type=jax.ShapeDtypeStruct((BATCH, SEQ, D_MODEL), jnp.float32),
    mesh=_mesh,
    compiler_params=pltpu.CompilerParams(use_tc_tiling_on_sc=True),
    scratch_types=[
        pltpu.VMEM((CHUNK_S, D_MODEL), jnp.float32),
        pltpu.VMEM((CHUNK_S, D_MODEL), jnp.float32),
        pltpu.VMEM((CHUNK_S, D_MODEL), jnp.float32),
        pltpu.VMEM((CHUNK_S, D_MODEL), jnp.float32),
        pltpu.VMEM((CHUNK_S, D_MODEL), jnp.float32),
        pltpu.VMEM((CHUNK_S, D_MODEL), jnp.float32),
        pltpu.SemaphoreType.DMA,
        pltpu.SemaphoreType.DMA,
        pltpu.SemaphoreType.DMA,
        pltpu.SemaphoreType.DMA,
        pltpu.SemaphoreType.DMA,
        pltpu.SemaphoreType.DMA,
        pltpu.SemaphoreType.DMA,
        pltpu.SemaphoreType.DMA,
        pltpu.SemaphoreType.DMA,
        pltpu.SemaphoreType.DMA,
    ],
)
def _sc_add_pe(x_hbm, pe_hbm, out_hbm, pe0, pe1, x0, x1, x2, x3,
               ps0, ps1, ls0, ls1, ls2, ls3, ss0, ss1, ss2, ss3):
    wid = lax.axis_index("c") * 16 + lax.axis_index("s")
    s0 = wid * S_PER_W
    pe_bufs, x_bufs = (pe0, pe1), (x0, x1, x2, x3)
    psems, lsems, ssems = (ps0, ps1), (ls0, ls1, ls2, ls3), (ss0, ss1, ss2, ss3)

    def x_slc(j):
        i, b = divmod(j, BATCH)
        return (b, pl.ds(s0 + i * CHUNK_S, CHUNK_S), slice(None))

    def start_pe(i):
        src = pe_hbm.at[pl.ds(s0 + i * CHUNK_S, CHUNK_S), :]
        return pltpu.async_copy(src, pe_bufs[i & 1], psems[i & 1])

    def start_load(j):
        src = x_hbm.at[x_slc(j)]
        return pltpu.async_copy(src, x_bufs[j % 4], lsems[j % 4])

    def start_store(j):
        dst = out_hbm.at[x_slc(j)]
        return pltpu.async_copy(x_bufs[j % 4], dst, ssems[j % 4])

    pe_d = {0: start_pe(0)}
    ld = {0: start_load(0), 1: start_load(1), 2: start_load(2)}
    st = {}
    for j in range(NITER):
        p = j % 4
        i = j // BATCH
        if j % BATCH == 0:
            pe_d[i].wait()
            if i + 1 < CHUNKS:
                pe_d[i + 1] = start_pe(i + 1)
        ld[j].wait()
        if j + 3 < NITER:
            if j >= 1:
                st[j - 1].wait()
            ld[j + 3] = start_load(j + 3)

        @plsc.parallel_loop(0, CHUNK_W // 16, unroll=16)
        def _(jj, _pe=pe_bufs[i & 1], _x=x_bufs[p]):
            r, c = divmod(jj * 16, D_MODEL)
            plsc.addupdate(_x.at[r, pl.ds(c, 16)], _pe[r, pl.ds(c, 16)])

        st[j] = start_store(j)
    st[NITER - 4].wait()
    st[NITER - 3].wait()
    st[NITER - 2].wait()
    st[NITER - 1].wait()


def kernel(x, pe_table):
    return _sc_add_pe(x, pe_table)
